# Initial kernel scaffold; baseline (speedup 1.0000x reference)
import jax
import jax.numpy as jnp
from jax.experimental import pallas as pl


def kernel(query, reference_points, input_flatten, input_spatial_shapes, input_level_start_index, Wv, bv, Ws, bs, Wa, ba, Wo, bo):
    raise NotImplementedError("write your pallas kernel here")


# speedup vs baseline: 2425.3186x; 2425.3186x over previous
import functools

import jax
import jax.numpy as jnp
import numpy as np
from jax import lax
from jax.experimental import pallas as pl
from jax.experimental.pallas import tpu as pltpu
from jax.experimental.pallas import tpu_sc as plsc

N = 2
LQ = 4096
DM = 256
NHEAD = 8
DH = DM // NHEAD
NLVL = 4
NPTS = 4
SHAPES = (4096, 2048, 1024, 512)
STARTS = (0, 4096, 6144, 7168)
LEN_IN = 7680

NQ = N * LQ
NQH = NQ * NHEAD
NROWS_V = N * LEN_IN * NHEAD

_NC, _NS = 2, 16
_NW = _NC * _NS
_ROWS_PER_W = NQH // _NW
_CB = 64
_NCHUNK = _ROWS_PER_W // _CB
_SPC = _CB * NLVL * NPTS
_IDX_ROWS = _SPC // 128

_lane = np.arange(128)
_lvl = (_lane // 4) % 4
_TVEC = np.array(SHAPES, np.float32)[_lvl]
_STARTH = (np.array(STARTS, np.int64)[_lvl] * NHEAD).astype(np.int32)
_HLANE = (_lane // 16).astype(np.int32)
_SEL = (np.arange(NLVL)[:, None] == _lvl[None, :]).astype(np.float32)
_BD = (_lane[:, None] // 16 == _lane[None, :] // 16).astype(np.float32)


def _mm_bias_body(x_ref, w_ref, b_ref, o_ref):
    o_ref[...] = (
        jnp.dot(x_ref[...], w_ref[...], preferred_element_type=jnp.float32, precision=jax.lax.Precision.HIGHEST)
        + b_ref[...]
    )


def _mm_bias(x, w_t, b, bm):
    m = x.shape[0]
    k = x.shape[1]
    n_out = w_t.shape[1]
    return pl.pallas_call(
        _mm_bias_body,
        grid=(m // bm,),
        in_specs=[
            pl.BlockSpec((bm, k), lambda i: (i, 0)),
            pl.BlockSpec((k, n_out), lambda i: (0, 0)),
            pl.BlockSpec((1, n_out), lambda i: (0, 0)),
        ],
        out_specs=pl.BlockSpec((bm, n_out), lambda i: (i, 0)),
        out_shape=jax.ShapeDtypeStruct((m, n_out), jnp.float32),
    )(x, w_t, b.reshape(1, n_out))


_PREP_BM = 1024


def _prep_body(q_ref, rp_ref, wst_ref, bs_ref, wat_ref, ba_ref,
               tv_ref, sh_ref, hl_ref, bd_ref,
               c0_ref, c1_ref, i0_ref, i1_ref):
    pid = pl.program_id(0)
    q = q_ref[...]
    off = jnp.dot(q, wst_ref[...], preferred_element_type=jnp.float32, precision=jax.lax.Precision.HIGHEST) + bs_ref[...]
    logits = jnp.dot(q, wat_ref[...], preferred_element_type=jnp.float32, precision=jax.lax.Precision.HIGHEST) + ba_ref[...]
    m = jnp.max(logits, axis=-1, keepdims=True)
    e = jnp.exp(logits - m)
    denom = jnp.dot(e, bd_ref[...], preferred_element_type=jnp.float32, precision=jax.lax.Precision.HIGHEST)
    aw = e / denom
    refb = rp_ref[...]
    tvec = tv_ref[...]
    loc = refb + off / tvec
    ix = jnp.clip(loc * tvec - 0.5, 0.0, tvec - 1.0)
    i0f = jnp.floor(ix)
    w = ix - i0f
    i0 = i0f.astype(jnp.int32)
    i1 = jnp.minimum(i0 + 1, tvec.astype(jnp.int32) - 1)
    nbase = (pid // (LQ // _PREP_BM)) * (LEN_IN * NHEAD)
    i0_ref[...] = nbase + sh_ref[...] + i0 * NHEAD + hl_ref[...]
    i1_ref[...] = nbase + sh_ref[...] + i1 * NHEAD + hl_ref[...]
    c0_ref[...] = aw * (1.0 - w)
    c1_ref[...] = aw * w


def _prep(q2, rp2, ws_t, bs, wa_t, ba):
    vec_spec = pl.BlockSpec((1, 128), lambda i: (0, 0))
    blk = pl.BlockSpec((_PREP_BM, 128), lambda i: (i, 0))
    f32 = jnp.float32
    return pl.pallas_call(
        _prep_body,
        grid=(NQ // _PREP_BM,),
        in_specs=[
            pl.BlockSpec((_PREP_BM, DM), lambda i: (i, 0)),
            blk,
            pl.BlockSpec((DM, 128), lambda i: (0, 0)),
            vec_spec,
            pl.BlockSpec((DM, 128), lambda i: (0, 0)),
            vec_spec,
            vec_spec, vec_spec, vec_spec,
            pl.BlockSpec((128, 128), lambda i: (0, 0)),
        ],
        out_specs=[blk, blk, blk, blk],
        out_shape=[
            jax.ShapeDtypeStruct((NQ, 128), f32),
            jax.ShapeDtypeStruct((NQ, 128), f32),
            jax.ShapeDtypeStruct((NQ, 128), jnp.int32),
            jax.ShapeDtypeStruct((NQ, 128), jnp.int32),
        ],
    )(q2, rp2, ws_t, bs.reshape(1, 128), wa_t, ba.reshape(1, 128),
      _TVEC.reshape(1, 128), _STARTH.reshape(1, 128), _HLANE.reshape(1, 128),
      _BD)


@functools.cache
def _get_sc_sample():
    mesh = plsc.VectorSubcoreMesh(core_axis_name="c", subcore_axis_name="s")

    @functools.partial(
        pl.kernel,
        mesh=mesh,
        compiler_params=pltpu.CompilerParams(
            needs_layout_passes=False, use_tc_tiling_on_sc=False),
        out_type=jax.ShapeDtypeStruct((NQH, DH), jnp.float32),
        scratch_types=[
            pltpu.VMEM((_IDX_ROWS, 128), jnp.int32),
            pltpu.VMEM((_IDX_ROWS, 128), jnp.int32),
            pltpu.VMEM((_SPC,), jnp.float32),
            pltpu.VMEM((_SPC,), jnp.float32),
            pltpu.VMEM((_SPC, DH), jnp.float32),
            pltpu.VMEM((_SPC, DH), jnp.float32),
            pltpu.VMEM((_CB, DH), jnp.float32),
            pltpu.SemaphoreType.DMA,
        ],
    )
    def _sc_sample(v_hbm, idx0_hbm, idx1_hbm, c0_hbm, c1_hbm, out_hbm,
                   idx0_v, idx1_v, c0_v, c1_v, rows0_v, rows1_v, out_v, sem):
        _sc_body(v_hbm, idx0_hbm, idx1_hbm, c0_hbm, c1_hbm, out_hbm,
                 idx0_v, idx1_v, c0_v, c1_v, rows0_v, rows1_v, out_v, sem)

    return _sc_sample


def _sc_body(v_hbm, idx0_hbm, idx1_hbm, c0_hbm, c1_hbm, out_hbm,
             idx0_v, idx1_v, c0_v, c1_v, rows0_v, rows1_v, out_v, sem):
    wid = lax.axis_index("s") * _NC + lax.axis_index("c")
    base_row_w = wid * _ROWS_PER_W

    def chunk(g, _):
        base_row = pl.multiple_of(base_row_w + g * _CB, _CB)
        base_samp = pl.multiple_of(base_row * (NLVL * NPTS), _SPC)
        idx_row0 = pl.multiple_of(base_samp // 128, _IDX_ROWS)
        pltpu.sync_copy(idx0_hbm.at[pl.ds(idx_row0, _IDX_ROWS)], idx0_v)
        pltpu.sync_copy(idx1_hbm.at[pl.ds(idx_row0, _IDX_ROWS)], idx1_v)
        pltpu.sync_copy(c0_hbm.at[pl.ds(base_samp, _SPC)], c0_v)
        pltpu.sync_copy(c1_hbm.at[pl.ds(base_samp, _SPC)], c1_v)
        descs = []
        for j in range(_IDX_ROWS):
            descs.append(pltpu.async_copy(
                v_hbm.at[idx0_v.at[j]], rows0_v.at[pl.ds(j * 128, 128)], sem))
            descs.append(pltpu.async_copy(
                v_hbm.at[idx1_v.at[j]], rows1_v.at[pl.ds(j * 128, 128)], sem))
        for d in descs:
            d.wait()

        def blend(r, _):
            s0 = r * (NLVL * NPTS)
            acc_lo = jnp.zeros((16,), jnp.float32)
            acc_hi = jnp.zeros((16,), jnp.float32)
            for k in range(NLVL * NPTS):
                sk = s0 + k
                iv = jnp.full((16,), 0, jnp.int32) + sk
                c0vec = plsc.load_gather(c0_v, [iv])
                c1vec = plsc.load_gather(c1_v, [iv])
                r0lo = rows0_v[sk, pl.ds(0, 16)]
                r0hi = rows0_v[sk, pl.ds(16, 16)]
                r1lo = rows1_v[sk, pl.ds(0, 16)]
                r1hi = rows1_v[sk, pl.ds(16, 16)]
                acc_lo = acc_lo + c0vec * r0lo + c1vec * r1lo
                acc_hi = acc_hi + c0vec * r0hi + c1vec * r1hi
            out_v[r, pl.ds(0, 16)] = acc_lo
            out_v[r, pl.ds(16, 16)] = acc_hi
            return 0

        lax.fori_loop(0, _CB, blend, 0)
        pltpu.sync_copy(out_v, out_hbm.at[pl.ds(base_row, _CB)])
        return 0

    lax.fori_loop(0, _NCHUNK, chunk, 0)


def kernel(query, reference_points, input_flatten, input_spatial_shapes,
           input_level_start_index, Wv, bv, Ws, bs, Wa, ba, Wo, bo):
    x = input_flatten.reshape(N * LEN_IN, DM)
    v = _mm_bias(x, Wv.T, bv, 1024)
    v_tab = v.reshape(NROWS_V, DH)

    q2 = query.reshape(NQ, DM)
    rp2 = reference_points.reshape(NQ, NLVL)
    rp128 = jnp.tile(jnp.repeat(rp2, NPTS, axis=1), (1, NHEAD))
    c0, c1, i0, i1 = _prep(q2, rp128, Ws.T, bs, Wa.T, ba)

    s = _get_sc_sample()(v_tab, i0.reshape(NQH * NLVL * NPTS // 128, 128),
                   i1.reshape(NQH * NLVL * NPTS // 128, 128),
                   c0.reshape(-1), c1.reshape(-1))

    out = _mm_bias(s.reshape(NQ, DM), Wo.T, bo, 1024)
    return out.reshape(N, LQ, DM)
